# baseline (speedup 1.0000x reference)
import functools

import jax
import jax.numpy as jnp
from jax import lax
from jax.experimental import pallas as pl
from jax.experimental.pallas import tpu as pltpu
from jax.experimental.pallas import tpu_sc as plsc

_L = 16


@functools.cache
def _build_sc_call(B, F, V, D):
    info = plsc.get_sparse_core_info()
    NC, NS = info.num_cores, info.num_subcores
    NW = NC * NS
    assert B % NW == 0 and D == _L and (F - 1) % _L == 0
    b_per_w = B // NW
    OUT_D = D + F - 1
    NF = (F - 1) // _L

    mesh = plsc.VectorSubcoreMesh(core_axis_name="c", subcore_axis_name="s")

    @functools.partial(
        pl.kernel,
        mesh=mesh,
        compiler_params=pltpu.CompilerParams(use_tc_tiling_on_sc=False),
        out_type=jax.ShapeDtypeStruct((B * OUT_D,), jnp.float32),
        scratch_types=[
            pltpu.VMEM((b_per_w,), jnp.int32),
            pltpu.VMEM((b_per_w * F,), jnp.float32),
            pltpu.VMEM((b_per_w, D), jnp.float32),
            pltpu.VMEM((b_per_w * OUT_D,), jnp.float32),
            pltpu.SemaphoreType.DMA,
        ],
    )
    def sc_kernel(x_hbm, idx_hbm, table_hbm, out_hbm, idx_v, x_v, emb_v, out_v, sem):
        wid = lax.axis_index("s") * NC + lax.axis_index("c")
        base = wid * b_per_w
        pltpu.sync_copy(idx_hbm.at[pl.ds(base, b_per_w)], idx_v)
        gather = pltpu.async_copy(table_hbm.at[idx_v], emb_v, sem)
        pltpu.sync_copy(x_hbm.at[pl.ds(base * F, b_per_w * F)], x_v)

        def feat_body(r, _):
            for j in range(NF):
                out_v[pl.ds(r * OUT_D + D + j * _L, _L)] = (
                    x_v[pl.ds(r * F + 1 + j * _L, _L)])
            return 0

        lax.fori_loop(0, b_per_w, feat_body, 0, unroll=4)
        gather.wait()

        def emb_body(r, _):
            out_v[pl.ds(r * OUT_D, _L)] = emb_v[r, :]
            return 0

        lax.fori_loop(0, b_per_w, emb_body, 0, unroll=4)
        pltpu.sync_copy(out_v, out_hbm.at[pl.ds(base * OUT_D, b_per_w * OUT_D)])

    return sc_kernel


def kernel(x, table):
    B, F = x.shape
    V, D = table.shape
    idx = x[:, 0].astype(jnp.int32)
    out_flat = _build_sc_call(B, F, V, D)(x.reshape(-1), idx, table)
    return out_flat.reshape(B, D + F - 1)
